# baseline (speedup 1.0000x reference)
import functools

import jax
import jax.numpy as jnp
from jax import lax
from jax.experimental import pallas as pl
from jax.experimental.pallas import tpu as pltpu
from jax.experimental.pallas import tpu_sc as plsc

VOCAB = 1000000
EMSIZE = 64
BATCH = 4096
HIST = 200
NW = 32
BB = BATCH // 128
NU = HIST
VL = 16

_mesh = plsc.VectorSubcoreMesh(core_axis_name="c", subcore_axis_name="s")


@functools.partial(
    pl.kernel,
    mesh=_mesh,
    out_type=jax.ShapeDtypeStruct((HIST, 8 * BB, 8 * 128), jnp.float32),
    scratch_types=[
        pltpu.VMEM((HIST, 128), jnp.int32),
        pltpu.VMEM((2, 8, VL, 128), jnp.float32),
        pltpu.VMEM((2, 8 * 8 * 128), jnp.float32),
        pltpu.VMEM((2, 8, VL), jnp.int32),
        pltpu.SemaphoreType.DMA((2,)),
        pltpu.SemaphoreType.DMA((2,)),
        pltpu.SemaphoreType.DMA,
    ],
    compiler_params=pltpu.CompilerParams(use_tc_tiling_on_sc=False,
                                         needs_layout_passes=False),
)
def _gather_kernel(idx_hbm, table_hbm, out_hbm, idx_v, grow_v, t_v, par_v,
                   gat_sem, wb_sem, idx_sem):
    wid = lax.axis_index("s") * 2 + lax.axis_index("c")

    pltpu.async_copy(
        idx_hbm.at[:, pl.ds(wid * 128, 128)], idx_v, idx_sem).wait()

    iota = lax.iota(jnp.int32, VL)

    def fire(h, b):
        for g in range(8):
            v = idx_v[h, pl.ds(g * VL, VL)]
            par_v[b, g] = (v & 1) * 64
            pltpu.async_copy(
                table_hbm.at[lax.shift_right_logical(v, 1)],
                grow_v.at[b, g], gat_sem.at[b])

    def gat_wait(b):
        for g in range(8):
            pltpu.make_async_copy(
                table_hbm.at[iota], grow_v.at[b, g], gat_sem.at[b]).wait()

    def transpose(b):
        @pl.loop(0, 8)
        def _eb(eb):
            e0 = eb * 8
            for g in range(8):
                colbase = par_v[b, g]
                for e in range(8):
                    vals = plsc.load_gather(
                        grow_v.at[b, g], [iota, colbase + (e0 + e)])
                    t_v[b, pl.ds(eb * 1024 + e * 128 + g * VL, VL)] = vals

    def wb(h, b):
        return [
            pltpu.make_async_copy(
                t_v.at[b, pl.ds(eb * 1024, 1024)],
                out_hbm.at[h, eb * BB + wid], wb_sem.at[b])
            for eb in range(8)
        ]

    def unit(h, b, *, first, last):
        if not last:
            fire(h + 1, 1 - b)
        gat_wait(b)
        if not first:
            for cp in wb(h - 2, b):
                cp.wait()
        transpose(b)
        for cp in wb(h, b):
            cp.start()

    fire(0, 0)
    unit(0, 0, first=True, last=False)
    unit(1, 1, first=True, last=False)

    @pl.loop(0, (NU - 4) // 2)
    def _steady(i):
        h0 = 2 + 2 * i
        unit(h0, 0, first=False, last=False)
        unit(h0 + 1, 1, first=False, last=False)

    unit(NU - 2, 0, first=False, last=False)
    unit(NU - 1, 1, first=False, last=True)
    for cp in wb(NU - 2, 0):
        cp.wait()
    for cp in wb(NU - 1, 1):
        cp.wait()


def kernel(input_variable, weight):
    idx_t = input_variable.astype(jnp.int32).T
    table_v = weight.reshape(VOCAB // 2, 128)
    out = _gather_kernel(idx_t, table_v)
    out = out.reshape(HIST, 8, BB, 8, 128)
    return out.transpose(2, 4, 0, 1, 3).reshape(BATCH, HIST, EMSIZE)


# speedup vs baseline: 1.5802x; 1.5802x over previous
import functools

import jax
import jax.numpy as jnp
from jax import lax
from jax.experimental import pallas as pl
from jax.experimental.pallas import tpu as pltpu
from jax.experimental.pallas import tpu_sc as plsc

VOCAB = 1000000
EMSIZE = 64
B_TOTAL = 4096 * 200
NW = 32
IDX_PER_W = B_TOTAL // NW
C = 640
G = IDX_PER_W // C

_mesh = plsc.VectorSubcoreMesh(core_axis_name="c", subcore_axis_name="s")


@functools.partial(
    pl.kernel,
    mesh=_mesh,
    out_type=jax.ShapeDtypeStruct((B_TOTAL, EMSIZE), jnp.float32),
    scratch_types=[
        pltpu.VMEM((IDX_PER_W,), jnp.int32),
        pltpu.VMEM((2, C, EMSIZE), jnp.float32),
        pltpu.SemaphoreType.DMA((2,)),
        pltpu.SemaphoreType.DMA((2,)),
        pltpu.SemaphoreType.DMA,
    ],
    compiler_params=pltpu.CompilerParams(use_tc_tiling_on_sc=False),
)
def _gather_kernel(idx_hbm, table_hbm, out_hbm, idx_v, rows_v,
                   gat_sem, out_sem, idx_sem):
    wid = lax.axis_index("s") * 2 + lax.axis_index("c")
    base = wid * IDX_PER_W

    pltpu.async_copy(
        idx_hbm.at[pl.ds(base, IDX_PER_W)], idx_v, idx_sem).wait()

    def fire(g, b):
        pltpu.async_copy(
            table_hbm.at[idx_v.at[pl.ds(g * C, C)]],
            rows_v.at[b], gat_sem.at[b])

    def gat_wait(b):
        pltpu.make_async_copy(
            table_hbm.at[idx_v.at[pl.ds(0, C)]],
            rows_v.at[b], gat_sem.at[b]).wait()

    def wb(g, b):
        o0 = base + g * C
        return pltpu.make_async_copy(
            rows_v.at[b], out_hbm.at[pl.ds(o0, C)], out_sem.at[b])

    def step(g, b, *, first, last):
        if not last:
            if not first:
                wb(g - 1, 1 - b).wait()
            fire(g + 1, 1 - b)
        gat_wait(b)
        wb(g, b).start()

    fire(0, 0)
    step(0, 0, first=True, last=False)

    @pl.loop(0, (G - 2) // 2)
    def _steady(i):
        g0 = 1 + 2 * i
        step(g0, 1, first=False, last=False)
        step(g0 + 1, 0, first=False, last=False)

    step(G - 1, 1, first=False, last=True)
    wb(G - 2, 0).wait()
    wb(G - 1, 1).wait()


def kernel(input_variable, weight):
    idx = input_variable.astype(jnp.int32).reshape(B_TOTAL)
    out = _gather_kernel(idx, weight)
    return out.reshape(input_variable.shape[0], input_variable.shape[1], EMSIZE)
